# Initial kernel scaffold; baseline (speedup 1.0000x reference)
import jax
import jax.numpy as jnp
from jax.experimental import pallas as pl


def kernel(h, batch, W1, b1, W2, b2, W3, b3):
    raise NotImplementedError("write your pallas kernel here")


# speedup vs baseline: 2.9266x; 2.9266x over previous
import functools

import jax
import jax.numpy as jnp
from jax import lax
from jax.experimental import pallas as pl
from jax.experimental.pallas import tpu as pltpu

NSEG = 512
R = 512


def _softplus(x):
    return jnp.logaddexp(x, 0.0)


def _body(batch_ref, h_ref, w1_ref, b1_ref, w2_ref, b2_ref, w3_ref, b3_ref,
          out_ref, acc_ref, *, nblocks):
    i = pl.program_id(0)

    @pl.when(i == 0)
    def _init():
        acc_ref[...] = jnp.zeros_like(acc_ref)

    y = _softplus(
        jnp.dot(h_ref[...], w1_ref[...], preferred_element_type=jnp.float32)
        + b1_ref[...])
    ids = batch_ref[0]
    seg_iota = lax.broadcasted_iota(jnp.int32, (NSEG, R), 0)
    onehot = (seg_iota == ids).astype(jnp.float32)
    acc_ref[...] += jnp.dot(onehot, y, preferred_element_type=jnp.float32)

    @pl.when(i == nblocks - 1)
    def _tail():
        z = _softplus(acc_ref[...])
        z = _softplus(
            jnp.dot(z, w2_ref[...], preferred_element_type=jnp.float32)
            + b2_ref[...])
        out_ref[...] = (
            jnp.dot(z, w3_ref[...], preferred_element_type=jnp.float32)
            + b3_ref[...])


def kernel(h, batch, W1, b1, W2, b2, W3, b3):
    n, dk = h.shape
    nblocks = pl.cdiv(n, R)
    batch3 = batch.astype(jnp.int32).reshape(nblocks, 1, R)
    grid = (nblocks,)
    out = pl.pallas_call(
        functools.partial(_body, nblocks=nblocks),
        grid=grid,
        in_specs=[
            pl.BlockSpec((1, 1, R), lambda i: (i, 0, 0)),
            pl.BlockSpec((R, dk), lambda i: (i, 0)),
            pl.BlockSpec(W1.shape, lambda i: (0, 0)),
            pl.BlockSpec((1, 256), lambda i: (0, 0)),
            pl.BlockSpec(W2.shape, lambda i: (0, 0)),
            pl.BlockSpec((1, 64), lambda i: (0, 0)),
            pl.BlockSpec(W3.shape, lambda i: (0, 0)),
            pl.BlockSpec((1, 1), lambda i: (0, 0)),
        ],
        out_specs=pl.BlockSpec((NSEG, 1), lambda i: (0, 0)),
        out_shape=jax.ShapeDtypeStruct((NSEG, 1), jnp.float32),
        scratch_shapes=[pltpu.VMEM((NSEG, 256), jnp.float32)],
        compiler_params=pltpu.CompilerParams(
            dimension_semantics=("arbitrary",)),
    )(batch3, h, W1, b1.reshape(1, 256), W2, b2.reshape(1, 64), W3,
      b3.reshape(1, 1))
    return out
